# Initial kernel scaffold; baseline (speedup 1.0000x reference)
import jax
import jax.numpy as jnp
from jax.experimental import pallas as pl


def kernel(base_plus, base_minus, signs):
    raise NotImplementedError("write your pallas kernel here")


# speedup vs baseline: 1.9516x; 1.9516x over previous
import math

import jax
import jax.numpy as jnp
from jax.experimental import pallas as pl

PHI = (1 + math.sqrt(5)) / 2
ALPHA = 1 / PHI
BETA = 1 / PHI ** 2

BM = 256


def _hv_kernel(signs_ref, bp_ref, bm_ref, out_ref):
    L = bp_ref.shape[0]
    m = (signs_ref[...] == 1).astype(jnp.float32)
    w = jnp.where(
        jax.lax.broadcasted_iota(jnp.int32, (L, 1), 0) == 0, ALPHA, ALPHA + BETA
    )
    diff = (bp_ref[...] - bm_ref[...]) * w
    const = jnp.sum(bm_ref[...] * w, axis=0, keepdims=True)
    hv = jnp.dot(m, diff, preferred_element_type=jnp.float32) + const
    norm = jnp.sqrt(jnp.sum(hv * hv, axis=1, keepdims=True))
    out_ref[...] = jnp.where(norm > 0, hv / jnp.maximum(norm, 1e-12), hv)


def kernel(base_plus, base_minus, signs):
    B, L = signs.shape
    D = base_plus.shape[1]
    return pl.pallas_call(
        _hv_kernel,
        grid=(B // BM,),
        in_specs=[
            pl.BlockSpec((BM, L), lambda i: (i, 0)),
            pl.BlockSpec((L, D), lambda i: (0, 0)),
            pl.BlockSpec((L, D), lambda i: (0, 0)),
        ],
        out_specs=pl.BlockSpec((BM, D), lambda i: (i, 0)),
        out_shape=jax.ShapeDtypeStruct((B, D), jnp.float32),
    )(signs, base_plus, base_minus)
